# baseline (speedup 1.0000x reference)
import jax
import jax.numpy as jnp
from jax import lax
from jax.experimental import pallas as pl
from jax.experimental.pallas import tpu as pltpu
from jax.experimental.pallas import tpu_sc as plsc

B = 1024
BAG = 50
BAGP = 56
D = 4096
NE = 4096
C = 2
DC = D // C
DC2 = DC // 2
NC = 2
NS = 16
NW = NC * NS
BW = B // NW
NV = 8
NB = DC2 // (16 * NV)
NR = 4

_MESH = plsc.VectorSubcoreMesh(core_axis_name="c", subcore_axis_name="s")


def _to_bf16_bits(acc):
    b = lax.bitcast_convert_type(acc, jnp.uint32)
    return (b + jnp.uint32(0x7FFF) + ((b >> jnp.uint32(16)) & jnp.uint32(1))
            ) >> jnp.uint32(16)


def _accumulate_bag(gbuf, rowring, slot):
    for h in range(NB):
        zero = jnp.zeros((16,), jnp.float32)
        init = (tuple(zero for _ in range(NV)), tuple(zero for _ in range(NV)))

        def lstep(l, accs, h=h):
            acc_lo, acc_hi = accs
            new_lo = []
            new_hi = []
            for v in range(NV):
                x = gbuf[l, pl.ds(h * (16 * NV) + v * 16, 16)]
                lo = lax.bitcast_convert_type(x << jnp.uint32(16), jnp.float32)
                hi = lax.bitcast_convert_type(x, jnp.float32)
                new_lo.append(acc_lo[v] + lo)
                new_hi.append(acc_hi[v] + hi)
            return (tuple(new_lo), tuple(new_hi))

        acc_lo, acc_hi = lax.fori_loop(0, BAG, lstep, init)
        for v in range(NV):
            word = (_to_bf16_bits(acc_hi[v]) << jnp.uint32(16)) | \
                _to_bf16_bits(acc_lo[v])
            rowring[slot, pl.ds(h * (16 * NV) + v * 16, 16)] = word


def _sc_body(table2, idxg, out3, idx_l, gbuf0, gbuf1, rowring, sem0, sem1,
             semw):
    w = lax.axis_index("s") * NC + lax.axis_index("c")
    base = w * BW
    pltpu.sync_copy(idxg.at[w], idx_l)

    def gather(c, j, gbuf, sem):
        return pltpu.async_copy(
            table2.at[idx_l.at[c, j, pl.ds(0, BAG)]], gbuf, sem)

    def gather_wait(c, j, gbuf, sem):
        pltpu.make_async_copy(
            table2.at[idx_l.at[c, j, pl.ds(0, BAG)]], gbuf, sem).wait()

    def write_row(c, j):
        slot = j & (NR - 1)
        pltpu.async_copy(rowring.at[slot], out3.at[c, base + j], semw)

    def write_wait(c, j):
        slot = j & (NR - 1)
        pltpu.make_async_copy(rowring.at[slot], out3.at[c, base + j],
                              semw).wait()

    for c in range(C):
        gather(c, 0, gbuf0, sem0)

        def pair(p, carry, c=c):
            j = p * 2
            gather_wait(c, j, gbuf0, sem0)
            gather(c, j + 1, gbuf1, sem1)

            @pl.when(p >= NR // 2)
            def _():
                write_wait(c, j - NR)
                write_wait(c, j - NR + 1)

            _accumulate_bag(gbuf0, rowring, j & (NR - 1))
            write_row(c, j)
            gather_wait(c, j + 1, gbuf1, sem1)

            @pl.when(p < (BW // 2 - 1))
            def _():
                gather(c, j + 2, gbuf0, sem0)

            _accumulate_bag(gbuf1, rowring, (j + 1) & (NR - 1))
            write_row(c, j + 1)
            return carry

        lax.fori_loop(0, BW // 2, pair, 0)
        for j in range(BW - NR, BW):
            write_wait(c, j)


_sc_call = pl.kernel(
    _sc_body,
    out_type=jax.ShapeDtypeStruct((C, B, DC2), jnp.uint32),
    mesh=_MESH,
    scratch_types=[
        pltpu.VMEM((C, BW, BAGP), jnp.int32),
        pltpu.VMEM((BAG, DC2), jnp.uint32),
        pltpu.VMEM((BAG, DC2), jnp.uint32),
        pltpu.VMEM((NR, DC2), jnp.uint32),
        pltpu.SemaphoreType.DMA,
        pltpu.SemaphoreType.DMA,
        pltpu.SemaphoreType.DMA,
    ],
)


@jax.jit
def kernel(codes, table):
    codes = codes.astype(jnp.int32)
    tb = table.astype(jnp.bfloat16).reshape(NE * C, DC2, 2)
    table2 = lax.bitcast_convert_type(tb, jnp.uint32)
    cp = jnp.pad(codes, ((0, 0), (0, BAGP - BAG))).reshape(NW, BW, BAGP)
    cvec = jnp.arange(C, dtype=jnp.int32)
    idxg = cp[:, None] * C + cvec[None, :, None, None]
    out3 = _sc_call(table2, idxg)
    outb = lax.bitcast_convert_type(out3, jnp.bfloat16)
    out = outb.reshape(C, B, DC).astype(jnp.float32)
    return out.transpose(1, 0, 2).reshape(B, D)


# speedup vs baseline: 11.3716x; 11.3716x over previous
import jax
import jax.numpy as jnp
from jax import lax
from jax.experimental import pallas as pl
from jax.experimental.pallas import tpu as pltpu
from jax.experimental.pallas import tpu_sc as plsc

B = 1024
BAG = 50
BAGP = 56
D = 4096
NE = 4096
C = 2
DC = D // C
DC2 = DC // 2
NC = 2
NS = 16
NW = NC * NS
BW = B // NW
NV = 8
NB = DC2 // (16 * NV)
NR = 4

_MESH = plsc.VectorSubcoreMesh(core_axis_name="c", subcore_axis_name="s")


def _to_bf16_bits(acc):
    b = lax.bitcast_convert_type(acc, jnp.uint32)
    return (b + jnp.uint32(0x7FFF) + ((b >> jnp.uint32(16)) & jnp.uint32(1))
            ) >> jnp.uint32(16)


def _accumulate_bag(gbuf, rowring, slot):
    for h in range(NB):
        zero = jnp.zeros((16,), jnp.float32)
        init = (tuple(zero for _ in range(NV)), tuple(zero for _ in range(NV)))

        def lstep(l, accs, h=h):
            acc_lo, acc_hi = accs
            new_lo = []
            new_hi = []
            for v in range(NV):
                x = gbuf[l, pl.ds(h * (16 * NV) + v * 16, 16)]
                lo = lax.bitcast_convert_type(x << jnp.uint32(16), jnp.float32)
                hi = lax.bitcast_convert_type(x, jnp.float32)
                new_lo.append(acc_lo[v] + lo)
                new_hi.append(acc_hi[v] + hi)
            return (tuple(new_lo), tuple(new_hi))

        acc_lo, acc_hi = lax.fori_loop(0, BAG, lstep, init)
        for v in range(NV):
            word = (_to_bf16_bits(acc_hi[v]) << jnp.uint32(16)) | \
                _to_bf16_bits(acc_lo[v])
            rowring[slot, pl.ds(h * (16 * NV) + v * 16, 16)] = word


def _sc_body(table2, idxg, out3, idx_l, gbuf0, gbuf1, rowring, sem0, sem1,
             semw):
    w = lax.axis_index("s") * NC + lax.axis_index("c")
    base = w * BW
    pltpu.sync_copy(idxg.at[w], idx_l)

    def gather(c, j, gbuf, sem):
        return pltpu.async_copy(
            table2.at[idx_l.at[c, j, pl.ds(0, BAG)]], gbuf, sem)

    def gather_wait(c, j, gbuf, sem):
        pltpu.make_async_copy(
            table2.at[idx_l.at[c, j, pl.ds(0, BAG)]], gbuf, sem).wait()

    def write_row(c, j):
        slot = j & (NR - 1)
        pltpu.async_copy(rowring.at[slot], out3.at[c, base + j], semw)

    def write_wait(c, j):
        slot = j & (NR - 1)
        pltpu.make_async_copy(rowring.at[slot], out3.at[c, base + j],
                              semw).wait()

    for c in range(C):
        gather(c, 0, gbuf0, sem0)

        def pair(p, carry, c=c):
            j = p * 2
            gather_wait(c, j, gbuf0, sem0)
            gather(c, j + 1, gbuf1, sem1)

            @pl.when(p >= NR // 2)
            def _():
                write_wait(c, j - NR)
                write_wait(c, j - NR + 1)

            _accumulate_bag(gbuf0, rowring, j & (NR - 1))
            write_row(c, j)
            gather_wait(c, j + 1, gbuf1, sem1)

            @pl.when(p < (BW // 2 - 1))
            def _():
                gather(c, j + 2, gbuf0, sem0)

            _accumulate_bag(gbuf1, rowring, (j + 1) & (NR - 1))
            write_row(c, j + 1)
            return carry

        lax.fori_loop(0, BW // 2, pair, 0)
        for j in range(BW - NR, BW):
            write_wait(c, j)


_sc_call = pl.kernel(
    _sc_body,
    out_type=jax.ShapeDtypeStruct((C, B, DC2), jnp.uint32),
    mesh=_MESH,
    scratch_types=[
        pltpu.VMEM((C, BW, BAGP), jnp.int32),
        pltpu.VMEM((BAG, DC2), jnp.uint32),
        pltpu.VMEM((BAG, DC2), jnp.uint32),
        pltpu.VMEM((NR, DC2), jnp.uint32),
        pltpu.SemaphoreType.DMA,
        pltpu.SemaphoreType.DMA,
        pltpu.SemaphoreType.DMA,
    ],
)


@jax.jit
def kernel(codes, table):
    codes = codes.astype(jnp.int32)
    tbits = lax.bitcast_convert_type(table, jnp.uint32).reshape(NE, D // 2, 2)
    r16 = (tbits + jnp.uint32(0x7FFF) +
           ((tbits >> jnp.uint32(16)) & jnp.uint32(1))) >> jnp.uint32(16)
    words = (r16[..., 1] << jnp.uint32(16)) | r16[..., 0]
    table2 = words.reshape(NE * C, DC2)
    cp = jnp.pad(codes, ((0, 0), (0, BAGP - BAG))).reshape(NW, BW, BAGP)
    cvec = jnp.arange(C, dtype=jnp.int32)
    idxg = cp[:, None] * C + cvec[None, :, None, None]
    out3 = _sc_call(table2, idxg)
    lo = lax.bitcast_convert_type(out3 << jnp.uint32(16), jnp.float32)
    hi = lax.bitcast_convert_type(
        out3 & jnp.uint32(0xFFFF0000), jnp.float32)
    out = jnp.stack([lo, hi], axis=-1).reshape(C, B, DC)
    return out.transpose(1, 0, 2).reshape(B, D)
